# baseline (speedup 1.0000x reference)
import functools

import jax
import jax.numpy as jnp
import numpy as np
from jax import lax
from jax.experimental import pallas as pl
from jax.experimental.pallas import tpu as pltpu
from jax.experimental.pallas import tpu_sc as plsc

_NC = 2
_NS = 16
_NW = _NC * _NS

_FWD_BLOCK = 1024


def _copy_body(x_ref, o_ref):
    o_ref[...] = x_ref[...]


def _pallas_copy(x):
    b, d = x.shape
    blk = min(_FWD_BLOCK, b)
    return pl.pallas_call(
        _copy_body,
        out_shape=jax.ShapeDtypeStruct(x.shape, x.dtype),
        grid=(b // blk,),
        in_specs=[pl.BlockSpec((blk, d), lambda i: (i, 0))],
        out_specs=pl.BlockSpec((blk, d), lambda i: (i, 0)),
    )(x)


def _sc_gather(table, idxs):
    b = idxs.shape[0]
    b_per_w = b // _NW
    mesh = plsc.VectorSubcoreMesh(core_axis_name="c", subcore_axis_name="s")

    @functools.partial(
        pl.kernel,
        mesh=mesh,
        out_type=jax.ShapeDtypeStruct((b,), jnp.float32),
        scratch_types=[
            pltpu.VMEM((b_per_w,), jnp.int32),
            pltpu.VMEM((b_per_w,), jnp.float32),
            pltpu.SemaphoreType.DMA,
        ],
    )
    def k(table_hbm, idx_hbm, out_hbm, idx_v, val_v, sem):
        wid = lax.axis_index("s") * _NC + lax.axis_index("c")
        base = wid * b_per_w
        pltpu.sync_copy(idx_hbm.at[pl.ds(base, b_per_w)], idx_v)
        pltpu.async_copy(table_hbm.at[idx_v], val_v, sem).wait()
        pltpu.sync_copy(val_v, out_hbm.at[pl.ds(base, b_per_w)])

    return k(table, idxs)


def _scale_body(g_ref, s_ref, o_ref):
    o_ref[...] = g_ref[...] * s_ref[...]


def _tc_scale(g, scaling):
    b, d = g.shape
    blk = min(_FWD_BLOCK, b)
    return pl.pallas_call(
        _scale_body,
        out_shape=jax.ShapeDtypeStruct((b, d), g.dtype),
        grid=(b // blk,),
        in_specs=[
            pl.BlockSpec((blk, d), lambda i: (i, 0)),
            pl.BlockSpec((blk, 1), lambda i: (i, 0)),
        ],
        out_specs=pl.BlockSpec((blk, d), lambda i: (i, 0)),
    )(g, scaling.reshape(b, 1))


@jax.custom_vjp
def _gsl(x, idxs, grad_scaling_values):
    return _pallas_copy(x)


def _gsl_fwd(x, idxs, grad_scaling_values):
    return _pallas_copy(x), (idxs, grad_scaling_values)


def _gsl_bwd(res, g):
    idxs, grad_scaling_values = res
    scaling = _sc_gather(grad_scaling_values, idxs)
    gx = _tc_scale(g, scaling)
    return (
        gx,
        np.zeros(idxs.shape, dtype=jax.dtypes.float0),
        jnp.zeros_like(grad_scaling_values),
    )


_gsl.defvjp(_gsl_fwd, _gsl_bwd)


def kernel(input, idxs, grad_scaling_values):
    return _gsl(input, idxs, grad_scaling_values)
